# Initial kernel scaffold; baseline (speedup 1.0000x reference)
import jax
import jax.numpy as jnp
from jax.experimental import pallas as pl


def kernel(x, edge_index, batch, W1, b1, W2, b2, W3, b3):
    raise NotImplementedError("write your pallas kernel here")


# speedup vs baseline: 24.4989x; 24.4989x over previous
import functools

import jax
import jax.numpy as jnp
from jax import lax
from jax.experimental import pallas as pl
from jax.experimental.pallas import tpu as pltpu
from jax.experimental.pallas import tpu_sc as plsc

_N = 10000
_E = 320000
_D = 128
_H = 64
_C = 10
_G = 64
_ALPHA = 0.1

_NC = 2
_NS = 16
_NW = _NC * _NS
_EPT = _E // _NW
_GRP = 80
_NGRP = _EPT // _GRP
_INIT_ROWS = 1000
_INIT_TILES = _N // _INIT_ROWS


def _sc_mesh():
    return plsc.VectorSubcoreMesh(
        core_axis_name="c", subcore_axis_name="s",
        num_cores=_NC, num_subcores=_NS)


def _deg_body(dst_hbm, zeros_hbm, out_hbm, idx_v, ones_v, acc_sh):
    c = lax.axis_index("c")
    s = lax.axis_index("s")
    wid = c * _NS + s
    pltpu.sync_copy(dst_hbm.at[wid], idx_v)
    for i in range(_GRP // 16):
        ones_v[pl.ds(i * 16, 16)] = jnp.ones((16,), jnp.float32)

    @pl.when(s < _INIT_TILES)
    def _():
        pltpu.sync_copy(zeros_hbm.at[pl.ds(s * _INIT_ROWS, _INIT_ROWS)],
                        acc_sh.at[pl.ds(s * _INIT_ROWS, _INIT_ROWS)])

    plsc.subcore_barrier()

    def body(j, carry):
        pltpu.sync_copy(ones_v, acc_sh.at[idx_v.at[j]], add=True)
        return carry

    lax.fori_loop(0, _NGRP, body, 0)
    plsc.subcore_barrier()

    @pl.when(s < _INIT_TILES)
    def _():
        pltpu.sync_copy(acc_sh.at[pl.ds(s * _INIT_ROWS, _INIT_ROWS)],
                        out_hbm.at[c, pl.ds(s * _INIT_ROWS, _INIT_ROWS)])


def _make_deg_call():
    return pl.kernel(
        _deg_body,
        out_type=jax.ShapeDtypeStruct((_NC, _N), jnp.float32),
        mesh=_sc_mesh(),
        compiler_params=pltpu.CompilerParams(use_tc_tiling_on_sc=False),
        scratch_types=[
            pltpu.VMEM((_NGRP, _GRP), jnp.int32),
            pltpu.VMEM((_GRP,), jnp.float32),
            pltpu.VMEM_SHARED((_N,), jnp.float32),
        ],
    )


def _step_body(src_hbm, dst_hbm, g_hbm, zeros_hbm, out_hbm,
               idxs_v, idxd_v, rows_v, acc_sh):
    c = lax.axis_index("c")
    s = lax.axis_index("s")
    wid = c * _NS + s
    pltpu.sync_copy(src_hbm.at[wid], idxs_v)
    pltpu.sync_copy(dst_hbm.at[wid], idxd_v)

    @pl.when(jnp.logical_and(c == 0, s < _INIT_TILES))
    def _():
        pltpu.sync_copy(g_hbm.at[pl.ds(s * _INIT_ROWS, _INIT_ROWS)],
                        acc_sh.at[pl.ds(s * _INIT_ROWS, _INIT_ROWS)])

    @pl.when(jnp.logical_and(c == 1, s < _INIT_TILES))
    def _():
        pltpu.sync_copy(zeros_hbm.at[pl.ds(s * _INIT_ROWS, _INIT_ROWS)],
                        acc_sh.at[pl.ds(s * _INIT_ROWS, _INIT_ROWS)])

    plsc.subcore_barrier()

    def body(j, carry):
        pltpu.sync_copy(g_hbm.at[idxs_v.at[j]], rows_v)
        pltpu.sync_copy(rows_v, acc_sh.at[idxd_v.at[j]], add=True)
        return carry

    lax.fori_loop(0, _NGRP, body, 0)
    plsc.subcore_barrier()

    @pl.when(s < _INIT_TILES)
    def _():
        pltpu.sync_copy(acc_sh.at[pl.ds(s * _INIT_ROWS, _INIT_ROWS)],
                        out_hbm.at[c, pl.ds(s * _INIT_ROWS, _INIT_ROWS)])


def _make_step_call():
    return pl.kernel(
        _step_body,
        out_type=jax.ShapeDtypeStruct((_NC, _N, _H), jnp.float32),
        mesh=_sc_mesh(),
        compiler_params=pltpu.CompilerParams(use_tc_tiling_on_sc=False),
        scratch_types=[
            pltpu.VMEM((_NGRP, _GRP), jnp.int32),
            pltpu.VMEM((_NGRP, _GRP), jnp.int32),
            pltpu.VMEM((_GRP, _H), jnp.float32),
            pltpu.VMEM_SHARED((_N, _H), jnp.float32),
        ],
    )


def _mlp_body(x_ref, w1_ref, b1_ref, w2_ref, b2_ref, degt_ref,
              h0_ref, g0_ref):
    x = x_ref[...]
    h = lax.dot_general(x, w1_ref[...], (((1,), (1,)), ((), ())),
                        preferred_element_type=jnp.float32)
    h = jnp.maximum(h + b1_ref[...], 0.0)
    h = lax.dot_general(h, w2_ref[...], (((1,), (1,)), ((), ())),
                        preferred_element_type=jnp.float32) + b2_ref[...]
    deg = degt_ref[:, 0:1] + degt_ref[:, 1:2] + 1.0
    dinv = lax.rsqrt(deg)
    h0_ref[...] = h
    g0_ref[...] = h * dinv


def _mlp_call(x, w1, b1r, w2, b2r, degt):
    return pl.pallas_call(
        _mlp_body,
        out_shape=(
            jax.ShapeDtypeStruct((_N, _H), jnp.float32),
            jax.ShapeDtypeStruct((_N, _H), jnp.float32),
        ),
    )(x, w1, b1r, w2, b2r, degt)


def _blend_body(sp_ref, h0_ref, degt_ref, gout_ref):
    S = sp_ref[0:_N, :] + sp_ref[_N:2 * _N, :]
    deg = degt_ref[:, 0:1] + degt_ref[:, 1:2] + 1.0
    dinv = lax.rsqrt(deg)
    h1 = (1.0 - _ALPHA) * (dinv * S) + _ALPHA * h0_ref[...]
    gout_ref[...] = dinv * h1


def _blend_call(sp2, h0, degt):
    return pl.pallas_call(
        _blend_body,
        out_shape=jax.ShapeDtypeStruct((_N, _H), jnp.float32),
    )(sp2, h0, degt)


def _final_body(sp_ref, h0_ref, degt_ref, batch_ref, w3_ref, b3_ref, out_ref):
    S = sp_ref[0:_N, :] + sp_ref[_N:2 * _N, :]
    deg = degt_ref[:, 0:1] + degt_ref[:, 1:2] + 1.0
    dinv = lax.rsqrt(deg)
    h2 = (1.0 - _ALPHA) * (dinv * S) + _ALPHA * h0_ref[...]
    gr = lax.broadcasted_iota(jnp.int32, (1, _G), 1)
    onehot = (batch_ref[...] == gr).astype(jnp.float32)
    ssum = lax.dot_general(onehot, h2, (((0,), (0,)), ((), ())),
                           preferred_element_type=jnp.float32)
    cnt = lax.dot_general(onehot, jnp.ones((_N, 1), jnp.float32),
                          (((0,), (0,)), ((), ())),
                          preferred_element_type=jnp.float32)
    mean = ssum / jnp.maximum(cnt, 1.0)
    out_ref[...] = lax.dot_general(mean, w3_ref[...],
                                   (((1,), (1,)), ((), ())),
                                   preferred_element_type=jnp.float32) \
        + b3_ref[...]


def _final_call(sp2, h0, degt, batch_col, w3, b3r):
    return pl.pallas_call(
        _final_body,
        out_shape=jax.ShapeDtypeStruct((_G, _C), jnp.float32),
    )(sp2, h0, degt, batch_col, w3, b3r)


@jax.jit
def kernel(x, edge_index, batch, W1, b1, W2, b2, W3, b3):
    src3 = edge_index[0].reshape(_NW, _NGRP, _GRP)
    dst3 = edge_index[1].reshape(_NW, _NGRP, _GRP)
    zeros_rows = jnp.zeros((_N, _H), jnp.float32)
    zeros_n = jnp.zeros((_N,), jnp.float32)

    deg_call = _make_deg_call()
    step_call = _make_step_call()

    degp = deg_call(dst3, zeros_n)
    degt = degp.T

    h0, g0 = _mlp_call(x, W1, b1.reshape(1, _H), W2, b2.reshape(1, _H), degt)

    sp1 = step_call(src3, dst3, g0, zeros_rows)
    g1 = _blend_call(sp1.reshape(2 * _N, _H), h0, degt)

    sp2 = step_call(src3, dst3, g1, zeros_rows)
    out = _final_call(sp2.reshape(2 * _N, _H), h0, degt,
                      batch.reshape(_N, 1), W3, b3.reshape(1, _C))
    return out


# speedup vs baseline: 37.9286x; 1.5482x over previous
import functools

import jax
import jax.numpy as jnp
from jax import lax
from jax.experimental import pallas as pl
from jax.experimental.pallas import tpu as pltpu
from jax.experimental.pallas import tpu_sc as plsc

_N = 10000
_E = 320000
_D = 128
_H = 64
_C = 10
_G = 64
_ALPHA = 0.1

_NC = 2
_NS = 16
_NW = _NC * _NS
_EPT = _E // _NW
_GRP = 125
_NGRP = _EPT // _GRP
_NBUF = 8
_NSG = _NGRP // _NBUF
_INIT_ROWS = 1000
_INIT_TILES = _N // _INIT_ROWS


def _sc_mesh():
    return plsc.VectorSubcoreMesh(
        core_axis_name="c", subcore_axis_name="s",
        num_cores=_NC, num_subcores=_NS)


def _deg_body(dst_hbm, zeros_hbm, out_hbm, idx_v, ones_v, acc_sh, sem_s):
    c = lax.axis_index("c")
    s = lax.axis_index("s")
    wid = c * _NS + s
    pltpu.sync_copy(dst_hbm.at[wid], idx_v)
    for i in range(8):
        ones_v[pl.ds(i * 16, 16)] = jnp.ones((16,), jnp.float32)

    @pl.when(s < _INIT_TILES)
    def _():
        pltpu.sync_copy(zeros_hbm.at[pl.ds(s * _INIT_ROWS, _INIT_ROWS)],
                        acc_sh.at[pl.ds(s * _INIT_ROWS, _INIT_ROWS)])

    plsc.subcore_barrier()

    descs = [pltpu.async_copy(ones_v.at[pl.ds(0, _GRP)],
                              acc_sh.at[idx_v.at[j]], sem_s, add=True)
             for j in range(_NGRP)]
    for d in descs:
        d.wait()
    plsc.subcore_barrier()

    @pl.when(s < _INIT_TILES)
    def _():
        pltpu.sync_copy(acc_sh.at[pl.ds(s * _INIT_ROWS, _INIT_ROWS)],
                        out_hbm.at[c, pl.ds(s * _INIT_ROWS, _INIT_ROWS)])


def _make_deg_call():
    return pl.kernel(
        _deg_body,
        out_type=jax.ShapeDtypeStruct((_NC, _N), jnp.float32),
        mesh=_sc_mesh(),
        compiler_params=pltpu.CompilerParams(use_tc_tiling_on_sc=False),
        scratch_types=[
            pltpu.VMEM((_NGRP, _GRP), jnp.int32),
            pltpu.VMEM((128,), jnp.float32),
            pltpu.VMEM_SHARED((_N,), jnp.float32),
            pltpu.SemaphoreType.DMA,
        ],
    )


def _step_body(src_hbm, dst_hbm, g_hbm, zeros_hbm, out_hbm,
               idxs_v, idxd_v, rows_v, acc_sh, sem_g, sem_s):
    c = lax.axis_index("c")
    s = lax.axis_index("s")
    wid = c * _NS + s
    pltpu.sync_copy(src_hbm.at[wid], idxs_v)
    pltpu.sync_copy(dst_hbm.at[wid], idxd_v)

    @pl.when(jnp.logical_and(c == 0, s < _INIT_TILES))
    def _():
        pltpu.sync_copy(g_hbm.at[pl.ds(s * _INIT_ROWS, _INIT_ROWS)],
                        acc_sh.at[pl.ds(s * _INIT_ROWS, _INIT_ROWS)])

    @pl.when(jnp.logical_and(c == 1, s < _INIT_TILES))
    def _():
        pltpu.sync_copy(zeros_hbm.at[pl.ds(s * _INIT_ROWS, _INIT_ROWS)],
                        acc_sh.at[pl.ds(s * _INIT_ROWS, _INIT_ROWS)])

    plsc.subcore_barrier()

    def body(sg, carry):
        g0 = sg * _NBUF
        gd = [pltpu.async_copy(g_hbm.at[idxs_v.at[g0 + b]], rows_v.at[b],
                               sem_g)
              for b in range(_NBUF)]
        for d in gd:
            d.wait()
        sd = [pltpu.async_copy(rows_v.at[b], acc_sh.at[idxd_v.at[g0 + b]],
                               sem_s, add=True)
              for b in range(_NBUF)]
        for d in sd:
            d.wait()
        return carry

    lax.fori_loop(0, _NSG, body, 0)
    plsc.subcore_barrier()

    @pl.when(s < _INIT_TILES)
    def _():
        pltpu.sync_copy(acc_sh.at[pl.ds(s * _INIT_ROWS, _INIT_ROWS)],
                        out_hbm.at[c, pl.ds(s * _INIT_ROWS, _INIT_ROWS)])


def _make_step_call():
    return pl.kernel(
        _step_body,
        out_type=jax.ShapeDtypeStruct((_NC, _N, _H), jnp.float32),
        mesh=_sc_mesh(),
        compiler_params=pltpu.CompilerParams(use_tc_tiling_on_sc=False),
        scratch_types=[
            pltpu.VMEM((_NGRP, _GRP), jnp.int32),
            pltpu.VMEM((_NGRP, _GRP), jnp.int32),
            pltpu.VMEM((_NBUF, _GRP, _H), jnp.float32),
            pltpu.VMEM_SHARED((_N, _H), jnp.float32),
            pltpu.SemaphoreType.DMA,
            pltpu.SemaphoreType.DMA,
        ],
    )


def _mlp_body(x_ref, w1_ref, b1_ref, w2_ref, b2_ref, degt_ref,
              h0_ref, g0_ref):
    x = x_ref[...]
    h = lax.dot_general(x, w1_ref[...], (((1,), (1,)), ((), ())),
                        preferred_element_type=jnp.float32)
    h = jnp.maximum(h + b1_ref[...], 0.0)
    h = lax.dot_general(h, w2_ref[...], (((1,), (1,)), ((), ())),
                        preferred_element_type=jnp.float32) + b2_ref[...]
    deg = degt_ref[:, 0:1] + degt_ref[:, 1:2] + 1.0
    dinv = lax.rsqrt(deg)
    h0_ref[...] = h
    g0_ref[...] = h * dinv


def _mlp_call(x, w1, b1r, w2, b2r, degt):
    return pl.pallas_call(
        _mlp_body,
        out_shape=(
            jax.ShapeDtypeStruct((_N, _H), jnp.float32),
            jax.ShapeDtypeStruct((_N, _H), jnp.float32),
        ),
    )(x, w1, b1r, w2, b2r, degt)


def _blend_body(sp_ref, h0_ref, degt_ref, gout_ref):
    S = sp_ref[0:_N, :] + sp_ref[_N:2 * _N, :]
    deg = degt_ref[:, 0:1] + degt_ref[:, 1:2] + 1.0
    dinv = lax.rsqrt(deg)
    h1 = (1.0 - _ALPHA) * (dinv * S) + _ALPHA * h0_ref[...]
    gout_ref[...] = dinv * h1


def _blend_call(sp2, h0, degt):
    return pl.pallas_call(
        _blend_body,
        out_shape=jax.ShapeDtypeStruct((_N, _H), jnp.float32),
    )(sp2, h0, degt)


def _final_body(sp_ref, h0_ref, degt_ref, batch_ref, w3_ref, b3_ref, out_ref):
    S = sp_ref[0:_N, :] + sp_ref[_N:2 * _N, :]
    deg = degt_ref[:, 0:1] + degt_ref[:, 1:2] + 1.0
    dinv = lax.rsqrt(deg)
    h2 = (1.0 - _ALPHA) * (dinv * S) + _ALPHA * h0_ref[...]
    gr = lax.broadcasted_iota(jnp.int32, (1, _G), 1)
    onehot = (batch_ref[...] == gr).astype(jnp.float32)
    ssum = lax.dot_general(onehot, h2, (((0,), (0,)), ((), ())),
                           preferred_element_type=jnp.float32)
    cnt = lax.dot_general(onehot, jnp.ones((_N, 1), jnp.float32),
                          (((0,), (0,)), ((), ())),
                          preferred_element_type=jnp.float32)
    mean = ssum / jnp.maximum(cnt, 1.0)
    out_ref[...] = lax.dot_general(mean, w3_ref[...],
                                   (((1,), (1,)), ((), ())),
                                   preferred_element_type=jnp.float32) \
        + b3_ref[...]


def _final_call(sp2, h0, degt, batch_col, w3, b3r):
    return pl.pallas_call(
        _final_body,
        out_shape=jax.ShapeDtypeStruct((_G, _C), jnp.float32),
    )(sp2, h0, degt, batch_col, w3, b3r)


@jax.jit
def kernel(x, edge_index, batch, W1, b1, W2, b2, W3, b3):
    src3 = edge_index[0].reshape(_NW, _NGRP, _GRP)
    dst3 = edge_index[1].reshape(_NW, _NGRP, _GRP)
    zeros_rows = jnp.zeros((_N, _H), jnp.float32)
    zeros_n = jnp.zeros((_N,), jnp.float32)

    deg_call = _make_deg_call()
    step_call = _make_step_call()

    degp = deg_call(dst3, zeros_n)
    degt = degp.T

    h0, g0 = _mlp_call(x, W1, b1.reshape(1, _H), W2, b2.reshape(1, _H), degt)

    sp1 = step_call(src3, dst3, g0, zeros_rows)
    g1 = _blend_call(sp1.reshape(2 * _N, _H), h0, degt)

    sp2 = step_call(src3, dst3, g1, zeros_rows)
    out = _final_call(sp2.reshape(2 * _N, _H), h0, degt,
                      batch.reshape(_N, 1), W3, b3.reshape(1, _C))
    return out


# speedup vs baseline: 42.4985x; 1.1205x over previous
import functools

import jax
import jax.numpy as jnp
from jax import lax
from jax.experimental import pallas as pl
from jax.experimental.pallas import tpu as pltpu
from jax.experimental.pallas import tpu_sc as plsc

_N = 10000
_E = 320000
_D = 128
_H = 64
_C = 10
_G = 64
_ALPHA = 0.1

_NC = 2
_NS = 16
_NW = _NC * _NS
_EPT = _E // _NW
_GRP = 125
_NGRP = _EPT // _GRP
_NBUF = 4
_NSG = _NGRP // _NBUF
_INIT_ROWS = 1000
_INIT_TILES = _N // _INIT_ROWS


def _sc_mesh():
    return plsc.VectorSubcoreMesh(
        core_axis_name="c", subcore_axis_name="s",
        num_cores=_NC, num_subcores=_NS)


def _deg_body(dst_hbm, zeros_hbm, out_hbm, idx_v, ones_v, acc_sh, sem_s):
    c = lax.axis_index("c")
    s = lax.axis_index("s")
    wid = c * _NS + s
    pltpu.sync_copy(dst_hbm.at[wid], idx_v)
    for i in range(8):
        ones_v[pl.ds(i * 16, 16)] = jnp.ones((16,), jnp.float32)

    @pl.when(s < _INIT_TILES)
    def _():
        pltpu.sync_copy(zeros_hbm.at[pl.ds(s * _INIT_ROWS, _INIT_ROWS)],
                        acc_sh.at[pl.ds(s * _INIT_ROWS, _INIT_ROWS)])

    plsc.subcore_barrier()

    descs = [pltpu.async_copy(ones_v.at[pl.ds(0, _GRP)],
                              acc_sh.at[idx_v.at[j]], sem_s, add=True)
             for j in range(_NGRP)]
    for d in descs:
        d.wait()
    plsc.subcore_barrier()

    @pl.when(s < _INIT_TILES)
    def _():
        pltpu.sync_copy(acc_sh.at[pl.ds(s * _INIT_ROWS, _INIT_ROWS)],
                        out_hbm.at[c, pl.ds(s * _INIT_ROWS, _INIT_ROWS)])


def _make_deg_call():
    return pl.kernel(
        _deg_body,
        out_type=jax.ShapeDtypeStruct((_NC, _N), jnp.float32),
        mesh=_sc_mesh(),
        compiler_params=pltpu.CompilerParams(use_tc_tiling_on_sc=False),
        scratch_types=[
            pltpu.VMEM((_NGRP, _GRP), jnp.int32),
            pltpu.VMEM((128,), jnp.float32),
            pltpu.VMEM_SHARED((_N,), jnp.float32),
            pltpu.SemaphoreType.DMA,
        ],
    )


def _step_body(src_hbm, dst_hbm, g_hbm, zeros_hbm, out_hbm,
               idxs_v, idxd_v, rows_v, acc_sh, sem_g, sem_g2, sem_s):
    c = lax.axis_index("c")
    s = lax.axis_index("s")
    wid = c * _NS + s
    pltpu.sync_copy(src_hbm.at[wid], idxs_v)
    pltpu.sync_copy(dst_hbm.at[wid], idxd_v)

    @pl.when(jnp.logical_and(c == 0, s < _INIT_TILES))
    def _():
        pltpu.sync_copy(g_hbm.at[pl.ds(s * _INIT_ROWS, _INIT_ROWS)],
                        acc_sh.at[pl.ds(s * _INIT_ROWS, _INIT_ROWS)])

    @pl.when(jnp.logical_and(c == 1, s < _INIT_TILES))
    def _():
        pltpu.sync_copy(zeros_hbm.at[pl.ds(s * _INIT_ROWS, _INIT_ROWS)],
                        acc_sh.at[pl.ds(s * _INIT_ROWS, _INIT_ROWS)])

    plsc.subcore_barrier()

    def _fire_gathers(sg, base, sem):
        for b in range(_NBUF):
            pltpu.async_copy(g_hbm.at[idxs_v.at[sg * _NBUF + b]],
                             rows_v.at[base + b], sem)

    def _wait_gathers(sg, base, sem):
        for b in range(_NBUF):
            pltpu.make_async_copy(g_hbm.at[idxs_v.at[sg * _NBUF + b]],
                                  rows_v.at[base + b], sem).wait()

    def _scatter(sg, base):
        sd = [pltpu.async_copy(rows_v.at[base + b],
                               acc_sh.at[idxd_v.at[sg * _NBUF + b]],
                               sem_s, add=True)
              for b in range(_NBUF)]
        for d in sd:
            d.wait()

    _fire_gathers(0, 0, sem_g)

    def pair(p, carry):
        sg_a = 2 * p
        sg_b = 2 * p + 1
        _wait_gathers(sg_a, 0, sem_g)
        _fire_gathers(sg_b, _NBUF, sem_g2)
        _scatter(sg_a, 0)
        _wait_gathers(sg_b, _NBUF, sem_g2)

        @pl.when(p < _NSG // 2 - 1)
        def _():
            _fire_gathers(sg_b + 1, 0, sem_g)

        _scatter(sg_b, _NBUF)
        return carry

    lax.fori_loop(0, _NSG // 2, pair, 0)
    plsc.subcore_barrier()

    @pl.when(s < _INIT_TILES)
    def _():
        pltpu.sync_copy(acc_sh.at[pl.ds(s * _INIT_ROWS, _INIT_ROWS)],
                        out_hbm.at[c, pl.ds(s * _INIT_ROWS, _INIT_ROWS)])


def _make_step_call():
    return pl.kernel(
        _step_body,
        out_type=jax.ShapeDtypeStruct((_NC, _N, _H), jnp.float32),
        mesh=_sc_mesh(),
        compiler_params=pltpu.CompilerParams(use_tc_tiling_on_sc=False),
        scratch_types=[
            pltpu.VMEM((_NGRP, _GRP), jnp.int32),
            pltpu.VMEM((_NGRP, _GRP), jnp.int32),
            pltpu.VMEM((2 * _NBUF, _GRP, _H), jnp.float32),
            pltpu.VMEM_SHARED((_N, _H), jnp.float32),
            pltpu.SemaphoreType.DMA,
            pltpu.SemaphoreType.DMA,
            pltpu.SemaphoreType.DMA,
        ],
    )


def _mlp_body(x_ref, w1_ref, b1_ref, w2_ref, b2_ref, degt_ref,
              h0_ref, g0_ref):
    x = x_ref[...]
    h = lax.dot_general(x, w1_ref[...], (((1,), (1,)), ((), ())),
                        preferred_element_type=jnp.float32)
    h = jnp.maximum(h + b1_ref[...], 0.0)
    h = lax.dot_general(h, w2_ref[...], (((1,), (1,)), ((), ())),
                        preferred_element_type=jnp.float32) + b2_ref[...]
    deg = degt_ref[:, 0:1] + degt_ref[:, 1:2] + 1.0
    dinv = lax.rsqrt(deg)
    h0_ref[...] = h
    g0_ref[...] = h * dinv


def _mlp_call(x, w1, b1r, w2, b2r, degt):
    return pl.pallas_call(
        _mlp_body,
        out_shape=(
            jax.ShapeDtypeStruct((_N, _H), jnp.float32),
            jax.ShapeDtypeStruct((_N, _H), jnp.float32),
        ),
    )(x, w1, b1r, w2, b2r, degt)


def _blend_body(sp_ref, h0_ref, degt_ref, gout_ref):
    S = sp_ref[0:_N, :] + sp_ref[_N:2 * _N, :]
    deg = degt_ref[:, 0:1] + degt_ref[:, 1:2] + 1.0
    dinv = lax.rsqrt(deg)
    h1 = (1.0 - _ALPHA) * (dinv * S) + _ALPHA * h0_ref[...]
    gout_ref[...] = dinv * h1


def _blend_call(sp2, h0, degt):
    return pl.pallas_call(
        _blend_body,
        out_shape=jax.ShapeDtypeStruct((_N, _H), jnp.float32),
    )(sp2, h0, degt)


def _final_body(sp_ref, h0_ref, degt_ref, batch_ref, w3_ref, b3_ref, out_ref):
    S = sp_ref[0:_N, :] + sp_ref[_N:2 * _N, :]
    deg = degt_ref[:, 0:1] + degt_ref[:, 1:2] + 1.0
    dinv = lax.rsqrt(deg)
    h2 = (1.0 - _ALPHA) * (dinv * S) + _ALPHA * h0_ref[...]
    gr = lax.broadcasted_iota(jnp.int32, (1, _G), 1)
    onehot = (batch_ref[...] == gr).astype(jnp.float32)
    ssum = lax.dot_general(onehot, h2, (((0,), (0,)), ((), ())),
                           preferred_element_type=jnp.float32)
    cnt = lax.dot_general(onehot, jnp.ones((_N, 1), jnp.float32),
                          (((0,), (0,)), ((), ())),
                          preferred_element_type=jnp.float32)
    mean = ssum / jnp.maximum(cnt, 1.0)
    out_ref[...] = lax.dot_general(mean, w3_ref[...],
                                   (((1,), (1,)), ((), ())),
                                   preferred_element_type=jnp.float32) \
        + b3_ref[...]


def _final_call(sp2, h0, degt, batch_col, w3, b3r):
    return pl.pallas_call(
        _final_body,
        out_shape=jax.ShapeDtypeStruct((_G, _C), jnp.float32),
    )(sp2, h0, degt, batch_col, w3, b3r)


@jax.jit
def kernel(x, edge_index, batch, W1, b1, W2, b2, W3, b3):
    src3 = edge_index[0].reshape(_NW, _NGRP, _GRP)
    dst3 = edge_index[1].reshape(_NW, _NGRP, _GRP)
    zeros_rows = jnp.zeros((_N, _H), jnp.float32)
    zeros_n = jnp.zeros((_N,), jnp.float32)

    deg_call = _make_deg_call()
    step_call = _make_step_call()

    degp = deg_call(dst3, zeros_n)
    degt = degp.T

    h0, g0 = _mlp_call(x, W1, b1.reshape(1, _H), W2, b2.reshape(1, _H), degt)

    sp1 = step_call(src3, dst3, g0, zeros_rows)
    g1 = _blend_call(sp1.reshape(2 * _N, _H), h0, degt)

    sp2 = step_call(src3, dst3, g1, zeros_rows)
    out = _final_call(sp2.reshape(2 * _N, _H), h0, degt,
                      batch.reshape(_N, 1), W3, b3.reshape(1, _C))
    return out
